# baseline (speedup 1.0000x reference)
import functools

import jax
import jax.numpy as jnp
from jax import lax
from jax.experimental import pallas as pl
from jax.experimental.pallas import tpu as pltpu
from jax.experimental.pallas import tpu_sc as plsc

B, H, N, M = 4, 16, 8192, 32
_NB = 512
_L = 16
_NV = N // _L
_ROWS = B * H
_NW = 32
_PASSES = ((0, 0x7FF), (11, 0x7FF), (22, 0x3FF))
_NBUCK = 2048
_SCAN_INCLUSIVE = True


def _sc_body(imp_hbm, thr_hbm, out_hbm, key_a, idx_a, key_b, idx_b,
             thr_v, out_v, hist0, hist1):
    nc = 2
    wid = lax.axis_index("s") * nc + lax.axis_index("c")
    h = wid % 16
    pltpu.sync_copy(thr_hbm.at[pl.ds(h * N, N)], thr_v)
    pltpu.sync_copy(imp_hbm.at[pl.ds(wid * N, N)], key_a.at[pl.ds(0, N)])
    pltpu.sync_copy(imp_hbm.at[pl.ds((wid + _NW) * N, N)], key_a.at[pl.ds(N, N)])
    iota = lax.broadcasted_iota(jnp.int32, (_L,), 0)
    zero16 = jnp.zeros((_L,), jnp.int32)
    minint = jnp.int32(-2**31)
    neg1 = jnp.int32(-1)

    def _hist_add(hist, d):
        cnt, last = plsc.scan_count(d)
        plsc.addupdate_scatter(hist, [d], cnt, mask=last)

    def _zero(hist, nbuck):
        def zr(j, _):
            hist[pl.ds(j * _L, _L)] = zero16
            return 0
        lax.fori_loop(0, nbuck // _L, zr, 0)

    def _excl_scan(hist, off, nbuck):
        def pb(j, carry):
            h16 = hist[pl.ds(off + j * _L, _L)]
            c = plsc.cumsum(h16)
            hist[pl.ds(off + j * _L, _L)] = c - h16 + carry
            return carry + jnp.sum(h16)
        lax.fori_loop(0, nbuck // _L, pb, jnp.int32(0))

    _zero(hist0, 2 * 2048)

    def prep(i, _):
        for rr in range(2):
            o = rr * N + i * _L
            u = key_a[pl.ds(o, _L)]
            m = lax.shift_right_arithmetic(u, 31)
            k = u ^ (m | minint) ^ neg1
            key_a[pl.ds(o, _L)] = k
            _hist_add(hist0, (k & 0x7FF) + rr * 2048)
        return 0

    lax.fori_loop(0, _NV, prep, 0)
    _zero(hist1, 2 * 2048)
    _excl_scan(hist0, 0, 2048)
    _excl_scan(hist0, 2048, 2048)

    def c0(i, _):
        for rr in range(2):
            o = rr * N + i * _L
            k = key_a[pl.ds(o, _L)]
            ix = i * _L + iota
            d = (k & 0x7FF) + rr * 2048
            cnt, last = plsc.scan_count(d)
            base = plsc.load_gather(hist0, [d])
            pos = base + cnt - 1 + rr * N
            plsc.store_scatter(key_b, [pos], k)
            plsc.store_scatter(idx_b, [pos], ix)
            plsc.addupdate_scatter(hist0, [d], cnt, mask=last)
            _hist_add(hist1, (lax.shift_right_logical(k, 11) & 0x7FF) + rr * 2048)
        return 0

    lax.fori_loop(0, _NV, c0, 0)
    _zero(hist0, 2 * 1024)
    _excl_scan(hist1, 0, 2048)
    _excl_scan(hist1, 2048, 2048)

    def c1(i, _):
        for rr in range(2):
            o = rr * N + i * _L
            k = key_b[pl.ds(o, _L)]
            ix = idx_b[pl.ds(o, _L)]
            d = (lax.shift_right_logical(k, 11) & 0x7FF) + rr * 2048
            cnt, last = plsc.scan_count(d)
            base = plsc.load_gather(hist1, [d])
            pos = base + cnt - 1 + rr * N
            plsc.store_scatter(key_a, [pos], k)
            plsc.store_scatter(idx_a, [pos], ix)
            plsc.addupdate_scatter(hist1, [d], cnt, mask=last)
            _hist_add(hist0, (lax.shift_right_logical(k, 22) & 0x3FF) + rr * 1024)
        return 0

    lax.fori_loop(0, _NV, c1, 0)
    _excl_scan(hist0, 0, 1024)
    _excl_scan(hist0, 1024, 1024)

    def c2(i, _):
        for rr in range(2):
            o = rr * N + i * _L
            k = key_a[pl.ds(o, _L)]
            ix = idx_a[pl.ds(o, _L)]
            d = (lax.shift_right_logical(k, 22) & 0x3FF) + rr * 1024
            cnt, last = plsc.scan_count(d)
            base = plsc.load_gather(hist0, [d])
            pos = base + cnt - 1
            t = plsc.load_gather(thr_v, [pos])
            plsc.store_scatter(out_v, [ix + rr * N], t)
            plsc.addupdate_scatter(hist0, [d], cnt, mask=last)
        return 0

    lax.fori_loop(0, _NV, c2, 0)
    pltpu.sync_copy(out_v.at[pl.ds(0, N)], out_hbm.at[pl.ds(wid * N, N)])
    pltpu.sync_copy(out_v.at[pl.ds(N, N)], out_hbm.at[pl.ds((wid + _NW) * N, N)])


def _sc_sorted_threshold(imp_flat, sim_threshold):
    mesh = plsc.VectorSubcoreMesh(core_axis_name="c", subcore_axis_name="s")
    f = pl.kernel(
        _sc_body,
        mesh=mesh,
        compiler_params=pltpu.CompilerParams(needs_layout_passes=False),
        out_type=jax.ShapeDtypeStruct((_ROWS * N,), jnp.float32),
        scratch_types=[
            pltpu.VMEM((2 * N,), jnp.int32),
            pltpu.VMEM((2 * N,), jnp.int32),
            pltpu.VMEM((2 * N,), jnp.int32),
            pltpu.VMEM((2 * N,), jnp.int32),
            pltpu.VMEM((N,), jnp.float32),
            pltpu.VMEM((2 * N,), jnp.float32),
            pltpu.VMEM((2 * _NBUCK,), jnp.int32),
            pltpu.VMEM((2 * _NBUCK,), jnp.int32),
        ],
    )
    return f(lax.bitcast_convert_type(imp_flat.reshape(-1), jnp.int32),
             sim_threshold.reshape(-1))


def _dense_body(sim_ref, st_ref, w_ref, s_ref, i_ref):
    acc = None
    for h in range(H):
        diff = sim_ref[0, h] - st_ref[0, h][None, :]
        diff = diff.astype(jnp.bfloat16).astype(jnp.float32)
        term = diff * w_ref[0, h]
        acc = term if acc is None else acc + term
    s_ref[0, 0, :] = 10.0 * jnp.tanh(jnp.max(acc, axis=0))
    i_ref[0, 0, :] = jnp.argmax(acc, axis=0).astype(jnp.int32)


def _dense_call(similarity, st, linear_w):
    nblk = N // _NB
    sim_t = jnp.swapaxes(similarity, 2, 3)
    s, i = pl.pallas_call(
        _dense_body,
        grid=(B, nblk),
        in_specs=[
            pl.BlockSpec((1, H, M, _NB), lambda b, n: (b, 0, 0, n)),
            pl.BlockSpec((1, H, _NB), lambda b, n: (b, 0, n)),
            pl.BlockSpec(memory_space=pltpu.SMEM),
        ],
        out_specs=[
            pl.BlockSpec((1, 1, _NB), lambda b, n: (b * nblk + n, 0, 0)),
            pl.BlockSpec((1, 1, _NB), lambda b, n: (b * nblk + n, 0, 0)),
        ],
        out_shape=[
            jax.ShapeDtypeStruct((B * nblk, 1, _NB), jnp.float32),
            jax.ShapeDtypeStruct((B * nblk, 1, _NB), jnp.int32),
        ],
    )(sim_t, st, _round_to_bf16(linear_w))
    return s.reshape(B, N), i.reshape(B, N)


def _round_to_bf16(x):
    u = lax.bitcast_convert_type(x, jnp.uint32)
    r = (u + jnp.uint32(0x7FFF) + ((u >> 16) & jnp.uint32(1))) & jnp.uint32(0xFFFF0000)
    return lax.bitcast_convert_type(r, jnp.float32)


def _sorted_threshold(importance, sim_threshold):
    st = _sc_sorted_threshold(importance.reshape(_ROWS, N), sim_threshold)
    return st.reshape(B, H, N)


def kernel(importance, similarity, compressed_map, sim_threshold, linear_w):
    del compressed_map
    st = _sorted_threshold(importance, sim_threshold)
    score, idx = _dense_call(similarity, st, linear_w)
    return score[..., None], idx, st[..., None]


# speedup vs baseline: 1.0073x; 1.0073x over previous
import functools

import jax
import jax.numpy as jnp
from jax import lax
from jax.experimental import pallas as pl
from jax.experimental.pallas import tpu as pltpu
from jax.experimental.pallas import tpu_sc as plsc

B, H, N, M = 4, 16, 8192, 32
_NB = 512
_L = 16
_NV = N // _L
_ROWS = B * H
_NW = 32
_PASSES = ((0, 0x7FF), (11, 0x7FF), (22, 0x3FF))
_NBUCK = 2048
_SCAN_INCLUSIVE = True


def _sc_body(imp_hbm, thr_hbm, out_hbm, key_a, idx_a, key_b, idx_b,
             thr_v, out_v, hist0, hist1):
    nc = 2
    wid = lax.axis_index("s") * nc + lax.axis_index("c")
    h = wid % 16
    pltpu.sync_copy(thr_hbm.at[pl.ds(h * N, N)], thr_v)
    iota = lax.broadcasted_iota(jnp.int32, (_L,), 0)
    zero16 = jnp.zeros((_L,), jnp.int32)
    minint = jnp.int32(-2**31)
    neg1 = jnp.int32(-1)

    def _hist_add(hist, d):
        cnt, last = plsc.scan_count(d)
        plsc.addupdate_scatter(hist, [d], cnt, mask=last)

    def _zero(hist, nbuck):
        def zr(j, _):
            hist[pl.ds(j * _L, _L)] = zero16
            return 0
        lax.fori_loop(0, nbuck // _L, zr, 0)

    def _excl_scan(hist, nbuck):
        def pb(j, carry):
            h16 = hist[pl.ds(j * _L, _L)]
            c = plsc.cumsum(h16)
            hist[pl.ds(j * _L, _L)] = c - h16 + carry
            return carry + jnp.sum(h16)
        lax.fori_loop(0, nbuck // _L, pb, jnp.int32(0))

    for r in range(2):
        row = wid + r * _NW
        pltpu.sync_copy(imp_hbm.at[pl.ds(row * N, N)], key_a)
        _zero(hist0, 2048)

        def prep(i, _):
            for u2 in range(2):
                o = (2 * i + u2) * _L
                u = key_a[pl.ds(o, _L)]
                m = lax.shift_right_arithmetic(u, 31)
                k = u ^ (m | minint) ^ neg1
                key_a[pl.ds(o, _L)] = k
                _hist_add(hist0, k & 0x7FF)
            return 0

        lax.fori_loop(0, _NV // 2, prep, 0)
        _zero(hist1, 2048)
        _excl_scan(hist0, 2048)

        def c0(i, _):
            for u2 in range(2):
                o = (2 * i + u2) * _L
                k = key_a[pl.ds(o, _L)]
                ix = o + iota
                d = k & 0x7FF
                cnt, last = plsc.scan_count(d)
                base = plsc.load_gather(hist0, [d])
                pos = base + cnt - 1
                plsc.store_scatter(key_b, [pos], k)
                plsc.store_scatter(idx_b, [pos], ix)
                plsc.addupdate_scatter(hist0, [d], cnt, mask=last)
                _hist_add(hist1, lax.shift_right_logical(k, 11) & 0x7FF)
            return 0

        lax.fori_loop(0, _NV // 2, c0, 0)
        _zero(hist0, 1024)
        _excl_scan(hist1, 2048)

        def c1(i, _):
            for u2 in range(2):
                o = (2 * i + u2) * _L
                k = key_b[pl.ds(o, _L)]
                ix = idx_b[pl.ds(o, _L)]
                d = lax.shift_right_logical(k, 11) & 0x7FF
                cnt, last = plsc.scan_count(d)
                base = plsc.load_gather(hist1, [d])
                pos = base + cnt - 1
                plsc.store_scatter(key_a, [pos], k)
                plsc.store_scatter(idx_a, [pos], ix)
                plsc.addupdate_scatter(hist1, [d], cnt, mask=last)
                _hist_add(hist0, lax.shift_right_logical(k, 22) & 0x3FF)
            return 0

        lax.fori_loop(0, _NV // 2, c1, 0)
        _excl_scan(hist0, 1024)

        def c2(i, _):
            for u2 in range(2):
                o = (2 * i + u2) * _L
                k = key_a[pl.ds(o, _L)]
                ix = idx_a[pl.ds(o, _L)]
                d = lax.shift_right_logical(k, 22) & 0x3FF
                cnt, last = plsc.scan_count(d)
                base = plsc.load_gather(hist0, [d])
                pos = base + cnt - 1
                t = plsc.load_gather(thr_v, [pos])
                plsc.store_scatter(out_v, [ix], t)
                plsc.addupdate_scatter(hist0, [d], cnt, mask=last)
            return 0

        lax.fori_loop(0, _NV // 2, c2, 0)
        pltpu.sync_copy(out_v, out_hbm.at[pl.ds(row * N, N)])


def _sc_sorted_threshold(imp_flat, sim_threshold):
    mesh = plsc.VectorSubcoreMesh(core_axis_name="c", subcore_axis_name="s")
    f = pl.kernel(
        _sc_body,
        mesh=mesh,
        compiler_params=pltpu.CompilerParams(needs_layout_passes=False),
        out_type=jax.ShapeDtypeStruct((_ROWS * N,), jnp.float32),
        scratch_types=[
            pltpu.VMEM((N,), jnp.int32),
            pltpu.VMEM((N,), jnp.int32),
            pltpu.VMEM((N,), jnp.int32),
            pltpu.VMEM((N,), jnp.int32),
            pltpu.VMEM((N,), jnp.float32),
            pltpu.VMEM((N,), jnp.float32),
            pltpu.VMEM((_NBUCK,), jnp.int32),
            pltpu.VMEM((_NBUCK,), jnp.int32),
        ],
    )
    return f(lax.bitcast_convert_type(imp_flat.reshape(-1), jnp.int32),
             sim_threshold.reshape(-1))


def _dense_body(sim_ref, st_ref, w_ref, s_ref, i_ref):
    acc = None
    for h in range(H):
        diff = sim_ref[0, h] - st_ref[0, h][None, :]
        diff = diff.astype(jnp.bfloat16).astype(jnp.float32)
        term = diff * w_ref[0, h]
        acc = term if acc is None else acc + term
    s_ref[0, 0, :] = 10.0 * jnp.tanh(jnp.max(acc, axis=0))
    i_ref[0, 0, :] = jnp.argmax(acc, axis=0).astype(jnp.int32)


def _dense_call(similarity, st, linear_w):
    nblk = N // _NB
    sim_t = jnp.swapaxes(similarity, 2, 3)
    s, i = pl.pallas_call(
        _dense_body,
        grid=(B, nblk),
        in_specs=[
            pl.BlockSpec((1, H, M, _NB), lambda b, n: (b, 0, 0, n)),
            pl.BlockSpec((1, H, _NB), lambda b, n: (b, 0, n)),
            pl.BlockSpec(memory_space=pltpu.SMEM),
        ],
        out_specs=[
            pl.BlockSpec((1, 1, _NB), lambda b, n: (b * nblk + n, 0, 0)),
            pl.BlockSpec((1, 1, _NB), lambda b, n: (b * nblk + n, 0, 0)),
        ],
        out_shape=[
            jax.ShapeDtypeStruct((B * nblk, 1, _NB), jnp.float32),
            jax.ShapeDtypeStruct((B * nblk, 1, _NB), jnp.int32),
        ],
    )(sim_t, st, _round_to_bf16(linear_w))
    return s.reshape(B, N), i.reshape(B, N)


def _round_to_bf16(x):
    u = lax.bitcast_convert_type(x, jnp.uint32)
    r = (u + jnp.uint32(0x7FFF) + ((u >> 16) & jnp.uint32(1))) & jnp.uint32(0xFFFF0000)
    return lax.bitcast_convert_type(r, jnp.float32)


def _sorted_threshold(importance, sim_threshold):
    st = _sc_sorted_threshold(importance.reshape(_ROWS, N), sim_threshold)
    return st.reshape(B, H, N)


def kernel(importance, similarity, compressed_map, sim_threshold, linear_w):
    del compressed_map
    st = _sorted_threshold(importance, sim_threshold)
    score, idx = _dense_call(similarity, st, linear_w)
    return score[..., None], idx, st[..., None]


# speedup vs baseline: 1.0979x; 1.0899x over previous
import jax
import jax.numpy as jnp
from jax import lax
from jax.experimental import pallas as pl
from jax.experimental.pallas import tpu as pltpu
from jax.experimental.pallas import tpu_sc as plsc

B, H, N, M = 4, 16, 8192, 32
_NB = 1024
_L = 16
_NV = N // _L
_ROWS = B * H
_NW = 32
_NBUCK = 2048


def _sc_body(imp_hbm, thr_hbm, out_hbm, key_a, idx_a, key_b, idx_b,
             thr_v, out_v, hist0, hist1):
    nc = 2
    wid = lax.axis_index("s") * nc + lax.axis_index("c")
    h = wid % 16
    pltpu.sync_copy(thr_hbm.at[pl.ds(h * N, N)], thr_v)
    iota = lax.broadcasted_iota(jnp.int32, (_L,), 0)
    zero16 = jnp.zeros((_L,), jnp.int32)
    minint = jnp.int32(-2**31)
    neg1 = jnp.int32(-1)

    def _hist_add(hist, d):
        cnt, last = plsc.scan_count(d)
        plsc.addupdate_scatter(hist, [d], cnt, mask=last)

    def _zero(hist, nbuck):
        def zr(j, _):
            hist[pl.ds(j * _L, _L)] = zero16
            return 0
        lax.fori_loop(0, nbuck // _L, zr, 0)

    def _excl_scan(hist, nbuck):
        def pb(j, carry):
            h16 = hist[pl.ds(j * _L, _L)]
            c = plsc.cumsum(h16)
            hist[pl.ds(j * _L, _L)] = c - h16 + carry
            return carry + jnp.sum(h16)
        lax.fori_loop(0, nbuck // _L, pb, jnp.int32(0))

    for r in range(2):
        row = wid + r * _NW
        pltpu.sync_copy(imp_hbm.at[pl.ds(row * N, N)], key_a)
        _zero(hist0, 2048)

        def prep(i, _):
            for u2 in range(2):
                o = (2 * i + u2) * _L
                u = key_a[pl.ds(o, _L)]
                m = lax.shift_right_arithmetic(u, 31)
                k = u ^ (m | minint) ^ neg1
                key_a[pl.ds(o, _L)] = k
                _hist_add(hist0, k & 0x7FF)
            return 0

        lax.fori_loop(0, _NV // 2, prep, 0)
        _zero(hist1, 2048)
        _excl_scan(hist0, 2048)

        def c0(i, _):
            for u2 in range(2):
                o = (2 * i + u2) * _L
                k = key_a[pl.ds(o, _L)]
                ix = o + iota
                d = k & 0x7FF
                cnt, last = plsc.scan_count(d)
                base = plsc.load_gather(hist0, [d])
                pos = base + cnt - 1
                plsc.store_scatter(key_b, [pos], k)
                plsc.store_scatter(idx_b, [pos], ix)
                plsc.addupdate_scatter(hist0, [d], cnt, mask=last)
                _hist_add(hist1, lax.shift_right_logical(k, 11) & 0x7FF)
            return 0

        lax.fori_loop(0, _NV // 2, c0, 0)
        _zero(hist0, 1024)
        _excl_scan(hist1, 2048)

        def c1(i, _):
            for u2 in range(2):
                o = (2 * i + u2) * _L
                k = key_b[pl.ds(o, _L)]
                ix = idx_b[pl.ds(o, _L)]
                d = lax.shift_right_logical(k, 11) & 0x7FF
                cnt, last = plsc.scan_count(d)
                base = plsc.load_gather(hist1, [d])
                pos = base + cnt - 1
                plsc.store_scatter(key_a, [pos], k)
                plsc.store_scatter(idx_a, [pos], ix)
                plsc.addupdate_scatter(hist1, [d], cnt, mask=last)
                _hist_add(hist0, lax.shift_right_logical(k, 22) & 0x3FF)
            return 0

        lax.fori_loop(0, _NV // 2, c1, 0)
        _excl_scan(hist0, 1024)

        def c2(i, _):
            for u2 in range(2):
                o = (2 * i + u2) * _L
                k = key_a[pl.ds(o, _L)]
                ix = idx_a[pl.ds(o, _L)]
                d = lax.shift_right_logical(k, 22) & 0x3FF
                cnt, last = plsc.scan_count(d)
                base = plsc.load_gather(hist0, [d])
                pos = base + cnt - 1
                t = plsc.load_gather(thr_v, [pos])
                plsc.store_scatter(out_v, [ix], t)
                plsc.addupdate_scatter(hist0, [d], cnt, mask=last)
            return 0

        lax.fori_loop(0, _NV // 2, c2, 0)
        pltpu.sync_copy(out_v, out_hbm.at[pl.ds(row * N, N)])


def _sc_sorted_threshold(imp_flat, sim_threshold):
    mesh = plsc.VectorSubcoreMesh(core_axis_name="c", subcore_axis_name="s")
    f = pl.kernel(
        _sc_body,
        mesh=mesh,
        compiler_params=pltpu.CompilerParams(needs_layout_passes=False),
        out_type=jax.ShapeDtypeStruct((_ROWS * N,), jnp.float32),
        scratch_types=[
            pltpu.VMEM((N,), jnp.int32),
            pltpu.VMEM((N,), jnp.int32),
            pltpu.VMEM((N,), jnp.int32),
            pltpu.VMEM((N,), jnp.int32),
            pltpu.VMEM((N,), jnp.float32),
            pltpu.VMEM((N,), jnp.float32),
            pltpu.VMEM((_NBUCK,), jnp.int32),
            pltpu.VMEM((_NBUCK,), jnp.int32),
        ],
    )
    return f(lax.bitcast_convert_type(imp_flat.reshape(-1), jnp.int32),
             sim_threshold.reshape(-1))


def _dense_body(sim_ref, st_ref, w_ref, s_ref, i_ref):
    acc = None
    for h in range(H):
        diff = sim_ref[0, h] - st_ref[0, h][None, :]
        diff = diff.astype(jnp.bfloat16).astype(jnp.float32)
        term = diff * w_ref[0, h]
        acc = term if acc is None else acc + term
    s_ref[0, 0, :] = 10.0 * jnp.tanh(jnp.max(acc, axis=0))
    i_ref[0, 0, :] = jnp.argmax(acc, axis=0).astype(jnp.int32)


def _dense_call(similarity, st, linear_w):
    nblk = N // _NB
    sim_t = jnp.swapaxes(similarity, 2, 3)
    s, i = pl.pallas_call(
        _dense_body,
        grid=(B, nblk),
        in_specs=[
            pl.BlockSpec((1, H, M, _NB), lambda b, n: (b, 0, 0, n)),
            pl.BlockSpec((1, H, _NB), lambda b, n: (b, 0, n)),
            pl.BlockSpec(memory_space=pltpu.SMEM),
        ],
        out_specs=[
            pl.BlockSpec((1, 1, _NB), lambda b, n: (b * nblk + n, 0, 0)),
            pl.BlockSpec((1, 1, _NB), lambda b, n: (b * nblk + n, 0, 0)),
        ],
        out_shape=[
            jax.ShapeDtypeStruct((B * nblk, 1, _NB), jnp.float32),
            jax.ShapeDtypeStruct((B * nblk, 1, _NB), jnp.int32),
        ],
    )(sim_t, st, _round_to_bf16(linear_w))
    return s.reshape(B, N), i.reshape(B, N)


def _round_to_bf16(x):
    u = lax.bitcast_convert_type(x, jnp.uint32)
    r = (u + jnp.uint32(0x7FFF) + ((u >> 16) & jnp.uint32(1))) & jnp.uint32(0xFFFF0000)
    return lax.bitcast_convert_type(r, jnp.float32)


def _sorted_threshold(importance, sim_threshold):
    st = _sc_sorted_threshold(importance.reshape(_ROWS, N), sim_threshold)
    return st.reshape(B, H, N)


def kernel(importance, similarity, compressed_map, sim_threshold, linear_w):
    del compressed_map
    st = _sorted_threshold(importance, sim_threshold)
    score, idx = _dense_call(similarity, st, linear_w)
    return score[..., None], idx, st[..., None]
